# baseline (speedup 1.0000x reference)
import functools

import jax
import jax.numpy as jnp
from jax import lax
from jax.experimental import pallas as pl
from jax.experimental.pallas import tpu as pltpu
from jax.experimental.pallas import tpu_sc as plsc

N = 10000
E = 160000
D = 256
DP = D // 2

NC = 2
NS = 16
W = NC * NS
L = 16

K = 64
PER_W = 5120
TAIL_W = E - (W - 1) * PER_W
CHUNKS = PER_W // K
NBUF = 4
UNROLL = 8

BLK = 1000


def _rne_bf16_bits(v):
  i = lax.bitcast_convert_type(v, jnp.int32)
  r = i + 0x7FFF + (lax.shift_right_logical(i, 16) & 1)
  return lax.shift_right_logical(r, 16)


def _norm_pack_body(xs_ref, xt_ref, os_ref, ot_ref):
  for x_ref, o_ref in ((xs_ref, os_ref), (xt_ref, ot_ref)):
    x = x_ref[...]
    ss = jnp.sum(x * x, axis=1, keepdims=True)
    xh = x * lax.rsqrt(jnp.maximum(ss, 1e-30))
    lo = _rne_bf16_bits(xh[:, :DP])
    hi = _rne_bf16_bits(xh[:, DP:])
    o_ref[...] = (lo & 0xFFFF) | lax.shift_left(hi, 16)


@jax.jit
def _norm_pack(x_source, x_target):
  return pl.pallas_call(
      _norm_pack_body,
      grid=(N // BLK,),
      in_specs=[
          pl.BlockSpec((BLK, D), lambda i: (i, 0)),
          pl.BlockSpec((BLK, D), lambda i: (i, 0)),
      ],
      out_specs=[
          pl.BlockSpec((BLK, DP), lambda i: (i, 0)),
          pl.BlockSpec((BLK, DP), lambda i: (i, 0)),
      ],
      out_shape=[
          jax.ShapeDtypeStruct((N, DP), jnp.int32),
          jax.ShapeDtypeStruct((N, DP), jnp.int32),
      ],
  )(x_source, x_target)


def _sc_body(xs_hbm, xt_hbm, edges_hbm, out_hbm,
             src_v, tgt_v, out_v, xs_buf, xt_buf, *sems):
  sems_xs = (sems[:NBUF], sems[NBUF:2 * NBUF])
  sems_xt = (sems[2 * NBUF:3 * NBUF], sems[3 * NBUF:])
  wid = lax.axis_index("s") * NC + lax.axis_index("c")
  base = wid * PER_W
  is_tail = wid == W - 1

  zero_i = jnp.zeros((L,), jnp.int32)

  @pl.when(jnp.logical_not(is_tail))
  def _():
    pltpu.sync_copy(edges_hbm.at[0, pl.ds(base, PER_W)], src_v)
    pltpu.sync_copy(edges_hbm.at[1, pl.ds(base, PER_W)], tgt_v)

  @pl.when(is_tail)
  def _():
    pltpu.sync_copy(edges_hbm.at[0, pl.ds(base, TAIL_W)],
                    src_v.at[pl.ds(0, TAIL_W)])
    pltpu.sync_copy(edges_hbm.at[1, pl.ds(base, TAIL_W)],
                    tgt_v.at[pl.ds(0, TAIL_W)])
    for j in range(TAIL_W, PER_W, L):
      src_v[pl.ds(j, L)] = zero_i
      tgt_v[pl.ds(j, L)] = zero_i

  H = K // 2

  def gather(c, b):
    for h in range(2):
      pltpu.async_copy(xs_hbm.at[src_v.at[pl.ds(c * K + h * H, H)]],
                       xs_buf.at[b, pl.ds(h * H, H)], sems_xs[h][b])
      pltpu.async_copy(xt_hbm.at[tgt_v.at[pl.ds(c * K + h * H, H)]],
                       xt_buf.at[b, pl.ds(h * H, H)], sems_xt[h][b])

  def wait(c, b):
    for h in range(2):
      pltpu.make_async_copy(xs_hbm.at[src_v.at[pl.ds(c * K + h * H, H)]],
                            xs_buf.at[b, pl.ds(h * H, H)], sems_xs[h][b]).wait()
      pltpu.make_async_copy(xt_hbm.at[tgt_v.at[pl.ds(c * K + h * H, H)]],
                            xt_buf.at[b, pl.ds(h * H, H)], sems_xt[h][b]).wait()

  for c in range(NBUF - 1):
    gather(c, c)

  zero = jnp.zeros((L,), jnp.float32)
  lane = lax.iota(jnp.int32, L)
  himask = jnp.full((L,), -65536, jnp.int32)

  def compute(c, b):
    xs = xs_buf.at[b]
    xt = xt_buf.at[b]
    for g in range(K // L):
      rows = lane + g * L

      def dbody(i, carry):
        dot0, dot1, col = carry
        for _ in range(UNROLL):
          sv = plsc.load_gather(xs, [rows, col])
          tv = plsc.load_gather(xt, [rows, col])
          sa = plsc.bitcast(lax.shift_left(sv, 16), jnp.float32)
          sb = plsc.bitcast(sv & himask, jnp.float32)
          ta = plsc.bitcast(lax.shift_left(tv, 16), jnp.float32)
          tb = plsc.bitcast(tv & himask, jnp.float32)
          dot0 = dot0 + sa * ta
          dot1 = dot1 + sb * tb
          col = (col + 1) & (DP - 1)
        return dot0, dot1, col

      dot0, dot1, _ = lax.fori_loop(
          0, DP // UNROLL, dbody, (zero, zero, lane))
      out_v[pl.ds(c * K + g * L, L)] = dot0 + dot1

  def chunk_quad(i, _):
    for b in range(NBUF):
      c = i * NBUF + b
      wait(c, b)

      @pl.when(c + NBUF - 1 < CHUNKS)
      def _():
        gather(c + NBUF - 1, (b + NBUF - 1) % NBUF)

      compute(c, b)
    return 0

  lax.fori_loop(0, CHUNKS // NBUF, chunk_quad, 0)

  @pl.when(jnp.logical_not(is_tail))
  def _():
    pltpu.sync_copy(out_v, out_hbm.at[pl.ds(base, PER_W)])

  @pl.when(is_tail)
  def _():
    pltpu.sync_copy(out_v.at[pl.ds(0, TAIL_W)],
                    out_hbm.at[pl.ds(base, TAIL_W)])


@jax.jit
def _cosine_sc(xsp, xtp, edges):
  return pl.kernel(
      _sc_body,
      out_type=jax.ShapeDtypeStruct((E,), jnp.float32),
      mesh=plsc.VectorSubcoreMesh(core_axis_name="c", subcore_axis_name="s"),
      compiler_params=pltpu.CompilerParams(
          use_tc_tiling_on_sc=False, needs_layout_passes=False),
      scratch_types=[
          pltpu.VMEM((PER_W,), jnp.int32),
          pltpu.VMEM((PER_W,), jnp.int32),
          pltpu.VMEM((PER_W,), jnp.float32),
          pltpu.VMEM((NBUF, K, DP), jnp.int32),
          pltpu.VMEM((NBUF, K, DP), jnp.int32),
      ] + [pltpu.SemaphoreType.DMA] * (4 * NBUF),
  )(xsp, xtp, edges)


def kernel(x_source, x_target, edge_label_index):
  edges = edge_label_index.astype(jnp.int32)
  xsp, xtp = _norm_pack(x_source, x_target)
  return _cosine_sc(xsp, xtp, edges)


# speedup vs baseline: 1.0879x; 1.0879x over previous
import functools

import jax
import jax.numpy as jnp
from jax import lax
from jax.experimental import pallas as pl
from jax.experimental.pallas import tpu as pltpu
from jax.experimental.pallas import tpu_sc as plsc

N = 10000
E = 160000
D = 256
DP = D // 2

NC = 2
NS = 16
L = 16

K = 64
FAST_CORE = 0
CH_FAST = 112
CH_SLOW = 48
PER_FAST = CH_FAST * K
PER_SLOW = CH_SLOW * K
E_PAD = NS * (PER_FAST + PER_SLOW)
NBUF = 4
UNROLL = 8

BLK = 1000


def _rne_bf16_bits(v):
  i = lax.bitcast_convert_type(v, jnp.int32)
  r = i + 0x7FFF + (lax.shift_right_logical(i, 16) & 1)
  return lax.shift_right_logical(r, 16)


def _norm_pack_body(xs_ref, xt_ref, os_ref, ot_ref):
  for x_ref, o_ref in ((xs_ref, os_ref), (xt_ref, ot_ref)):
    x = x_ref[...]
    ss = jnp.sum(x * x, axis=1, keepdims=True)
    xh = x * lax.rsqrt(jnp.maximum(ss, 1e-30))
    lo = _rne_bf16_bits(xh[:, :DP])
    hi = _rne_bf16_bits(xh[:, DP:])
    o_ref[...] = (lo & 0xFFFF) | lax.shift_left(hi, 16)


@jax.jit
def _norm_pack(x_source, x_target):
  return pl.pallas_call(
      _norm_pack_body,
      grid=(N // BLK,),
      in_specs=[
          pl.BlockSpec((BLK, D), lambda i: (i, 0)),
          pl.BlockSpec((BLK, D), lambda i: (i, 0)),
      ],
      out_specs=[
          pl.BlockSpec((BLK, DP), lambda i: (i, 0)),
          pl.BlockSpec((BLK, DP), lambda i: (i, 0)),
      ],
      out_shape=[
          jax.ShapeDtypeStruct((N, DP), jnp.int32),
          jax.ShapeDtypeStruct((N, DP), jnp.int32),
      ],
  )(x_source, x_target)


def _sc_body(xs_hbm, xt_hbm, src_hbm, tgt_hbm, out_hbm,
             src_v, tgt_v, out_v, xs_buf, xt_buf, *sems):
  sems_xs = sems[:NBUF]
  sems_xt = sems[NBUF:]
  cid = lax.axis_index("c")
  sid = lax.axis_index("s")
  on_fast = cid == FAST_CORE
  base = jnp.where(on_fast, sid * PER_FAST,
                   NS * PER_FAST + sid * PER_SLOW)
  n_quads = jnp.where(on_fast, CH_FAST // NBUF, CH_SLOW // NBUF)

  @pl.when(on_fast)
  def _():
    pltpu.sync_copy(src_hbm.at[pl.ds(base, PER_FAST)], src_v)
    pltpu.sync_copy(tgt_hbm.at[pl.ds(base, PER_FAST)], tgt_v)

  @pl.when(jnp.logical_not(on_fast))
  def _():
    pltpu.sync_copy(src_hbm.at[pl.ds(base, PER_SLOW)],
                    src_v.at[pl.ds(0, PER_SLOW)])
    pltpu.sync_copy(tgt_hbm.at[pl.ds(base, PER_SLOW)],
                    tgt_v.at[pl.ds(0, PER_SLOW)])

  def gather(c, b):
    pltpu.async_copy(xs_hbm.at[src_v.at[pl.ds(c * K, K)]],
                     xs_buf.at[b], sems_xs[b])
    pltpu.async_copy(xt_hbm.at[tgt_v.at[pl.ds(c * K, K)]],
                     xt_buf.at[b], sems_xt[b])

  def wait(c, b):
    pltpu.make_async_copy(xs_hbm.at[src_v.at[pl.ds(c * K, K)]],
                          xs_buf.at[b], sems_xs[b]).wait()
    pltpu.make_async_copy(xt_hbm.at[tgt_v.at[pl.ds(c * K, K)]],
                          xt_buf.at[b], sems_xt[b]).wait()

  for c in range(NBUF - 1):
    gather(c, c)

  zero = jnp.zeros((L,), jnp.float32)
  lane = lax.iota(jnp.int32, L)
  himask = jnp.full((L,), -65536, jnp.int32)

  def compute(c, b):
    xs = xs_buf.at[b]
    xt = xt_buf.at[b]
    for g in range(K // L):
      rows = lane + g * L

      def dbody(i, carry):
        dot0, dot1, col = carry
        for _ in range(UNROLL):
          sv = plsc.load_gather(xs, [rows, col])
          tv = plsc.load_gather(xt, [rows, col])
          sa = plsc.bitcast(lax.shift_left(sv, 16), jnp.float32)
          sb = plsc.bitcast(sv & himask, jnp.float32)
          ta = plsc.bitcast(lax.shift_left(tv, 16), jnp.float32)
          tb = plsc.bitcast(tv & himask, jnp.float32)
          dot0 = dot0 + sa * ta
          dot1 = dot1 + sb * tb
          col = (col + 1) & (DP - 1)
        return dot0, dot1, col

      dot0, dot1, _ = lax.fori_loop(
          0, DP // UNROLL, dbody, (zero, zero, lane))
      out_v[pl.ds(c * K + g * L, L)] = dot0 + dot1

  def chunk_quad(i, _):
    for b in range(NBUF):
      c = i * NBUF + b
      wait(c, b)

      @pl.when(c + NBUF - 1 < n_quads * NBUF)
      def _():
        gather(c + NBUF - 1, (b + NBUF - 1) % NBUF)

      compute(c, b)
    return 0

  lax.fori_loop(0, n_quads, chunk_quad, 0)

  @pl.when(on_fast)
  def _():
    pltpu.sync_copy(out_v, out_hbm.at[pl.ds(base, PER_FAST)])

  @pl.when(jnp.logical_not(on_fast))
  def _():
    pltpu.sync_copy(out_v.at[pl.ds(0, PER_SLOW)],
                    out_hbm.at[pl.ds(base, PER_SLOW)])


@jax.jit
def _cosine_sc(xsp, xtp, src, tgt):
  return pl.kernel(
      _sc_body,
      out_type=jax.ShapeDtypeStruct((E_PAD,), jnp.float32),
      mesh=plsc.VectorSubcoreMesh(core_axis_name="c", subcore_axis_name="s"),
      compiler_params=pltpu.CompilerParams(
          use_tc_tiling_on_sc=False, needs_layout_passes=False),
      scratch_types=[
          pltpu.VMEM((PER_FAST,), jnp.int32),
          pltpu.VMEM((PER_FAST,), jnp.int32),
          pltpu.VMEM((PER_FAST,), jnp.float32),
          pltpu.VMEM((NBUF, K, DP), jnp.int32),
          pltpu.VMEM((NBUF, K, DP), jnp.int32),
      ] + [pltpu.SemaphoreType.DMA] * (2 * NBUF),
  )(xsp, xtp, src, tgt)


def kernel(x_source, x_target, edge_label_index):
  src = edge_label_index[0].astype(jnp.int32)
  tgt = edge_label_index[1].astype(jnp.int32)
  pad = E_PAD - E
  src = jnp.concatenate([src, jnp.zeros((pad,), jnp.int32)])
  tgt = jnp.concatenate([tgt, jnp.zeros((pad,), jnp.int32)])
  xsp, xtp = _norm_pack(x_source, x_target)
  out = _cosine_sc(xsp, xtp, src, tgt)
  return out[:E]


# speedup vs baseline: 1.1068x; 1.0173x over previous
import functools

import jax
import jax.numpy as jnp
from jax import lax
from jax.experimental import pallas as pl
from jax.experimental.pallas import tpu as pltpu
from jax.experimental.pallas import tpu_sc as plsc

N = 10000
E = 160000
D = 256
DP = D // 2

NC = 2
NS = 16
L = 16

K = 64
FAST_CORE = 0
CH_FAST = 124
CH_SLOW = 36
PER_FAST = CH_FAST * K
PER_SLOW = CH_SLOW * K
E_PAD = NS * (PER_FAST + PER_SLOW)
NBUF = 4
UNROLL = 8

BLK = 1000


def _rne_bf16_bits(v):
  i = lax.bitcast_convert_type(v, jnp.int32)
  r = i + 0x7FFF + (lax.shift_right_logical(i, 16) & 1)
  return lax.shift_right_logical(r, 16)


def _norm_pack_body(xs_ref, xt_ref, os_ref, ot_ref):
  for x_ref, o_ref in ((xs_ref, os_ref), (xt_ref, ot_ref)):
    x = x_ref[...]
    ss = jnp.sum(x * x, axis=1, keepdims=True)
    xh = x * lax.rsqrt(jnp.maximum(ss, 1e-30))
    lo = _rne_bf16_bits(xh[:, :DP])
    hi = _rne_bf16_bits(xh[:, DP:])
    o_ref[...] = (lo & 0xFFFF) | lax.shift_left(hi, 16)


@jax.jit
def _norm_pack(x_source, x_target):
  return pl.pallas_call(
      _norm_pack_body,
      grid=(N // BLK,),
      in_specs=[
          pl.BlockSpec((BLK, D), lambda i: (i, 0)),
          pl.BlockSpec((BLK, D), lambda i: (i, 0)),
      ],
      out_specs=[
          pl.BlockSpec((BLK, DP), lambda i: (i, 0)),
          pl.BlockSpec((BLK, DP), lambda i: (i, 0)),
      ],
      out_shape=[
          jax.ShapeDtypeStruct((N, DP), jnp.int32),
          jax.ShapeDtypeStruct((N, DP), jnp.int32),
      ],
  )(x_source, x_target)


def _sc_body(xs_hbm, xt_hbm, src_hbm, tgt_hbm, out_hbm,
             src_v, tgt_v, out_v, xs_buf, xt_buf, *sems):
  sems_xs = sems[:NBUF]
  sems_xt = sems[NBUF:]
  cid = lax.axis_index("c")
  sid = lax.axis_index("s")
  on_fast = cid == FAST_CORE
  base = jnp.where(on_fast, sid * PER_FAST,
                   NS * PER_FAST + sid * PER_SLOW)
  n_quads = jnp.where(on_fast, CH_FAST // NBUF, CH_SLOW // NBUF)

  @pl.when(on_fast)
  def _():
    pltpu.sync_copy(src_hbm.at[pl.ds(base, PER_FAST)], src_v)
    pltpu.sync_copy(tgt_hbm.at[pl.ds(base, PER_FAST)], tgt_v)

  @pl.when(jnp.logical_not(on_fast))
  def _():
    pltpu.sync_copy(src_hbm.at[pl.ds(base, PER_SLOW)],
                    src_v.at[pl.ds(0, PER_SLOW)])
    pltpu.sync_copy(tgt_hbm.at[pl.ds(base, PER_SLOW)],
                    tgt_v.at[pl.ds(0, PER_SLOW)])

  def gather(c, b):
    pltpu.async_copy(xs_hbm.at[src_v.at[pl.ds(c * K, K)]],
                     xs_buf.at[b], sems_xs[b])
    pltpu.async_copy(xt_hbm.at[tgt_v.at[pl.ds(c * K, K)]],
                     xt_buf.at[b], sems_xt[b])

  def wait(c, b):
    pltpu.make_async_copy(xs_hbm.at[src_v.at[pl.ds(c * K, K)]],
                          xs_buf.at[b], sems_xs[b]).wait()
    pltpu.make_async_copy(xt_hbm.at[tgt_v.at[pl.ds(c * K, K)]],
                          xt_buf.at[b], sems_xt[b]).wait()

  for c in range(NBUF - 1):
    gather(c, c)

  zero = jnp.zeros((L,), jnp.float32)
  lane = lax.iota(jnp.int32, L)
  himask = jnp.full((L,), -65536, jnp.int32)

  def compute(c, b):
    xs = xs_buf.at[b]
    xt = xt_buf.at[b]
    for g in range(K // L):
      rows = lane + g * L

      def dbody(i, carry):
        dot0, dot1, col = carry
        for _ in range(UNROLL):
          sv = plsc.load_gather(xs, [rows, col])
          tv = plsc.load_gather(xt, [rows, col])
          sa = plsc.bitcast(lax.shift_left(sv, 16), jnp.float32)
          sb = plsc.bitcast(sv & himask, jnp.float32)
          ta = plsc.bitcast(lax.shift_left(tv, 16), jnp.float32)
          tb = plsc.bitcast(tv & himask, jnp.float32)
          dot0 = dot0 + sa * ta
          dot1 = dot1 + sb * tb
          col = (col + 1) & (DP - 1)
        return dot0, dot1, col

      dot0, dot1, _ = lax.fori_loop(
          0, DP // UNROLL, dbody, (zero, zero, lane))
      out_v[pl.ds(c * K + g * L, L)] = dot0 + dot1

  def chunk_quad(i, _):
    for b in range(NBUF):
      c = i * NBUF + b
      wait(c, b)

      @pl.when(c + NBUF - 1 < n_quads * NBUF)
      def _():
        gather(c + NBUF - 1, (b + NBUF - 1) % NBUF)

      compute(c, b)
    return 0

  lax.fori_loop(0, n_quads, chunk_quad, 0)

  @pl.when(on_fast)
  def _():
    pltpu.sync_copy(out_v, out_hbm.at[pl.ds(base, PER_FAST)])

  @pl.when(jnp.logical_not(on_fast))
  def _():
    pltpu.sync_copy(out_v.at[pl.ds(0, PER_SLOW)],
                    out_hbm.at[pl.ds(base, PER_SLOW)])


@jax.jit
def _cosine_sc(xsp, xtp, src, tgt):
  return pl.kernel(
      _sc_body,
      out_type=jax.ShapeDtypeStruct((E_PAD,), jnp.float32),
      mesh=plsc.VectorSubcoreMesh(core_axis_name="c", subcore_axis_name="s"),
      compiler_params=pltpu.CompilerParams(
          use_tc_tiling_on_sc=False, needs_layout_passes=False),
      scratch_types=[
          pltpu.VMEM((PER_FAST,), jnp.int32),
          pltpu.VMEM((PER_FAST,), jnp.int32),
          pltpu.VMEM((PER_FAST,), jnp.float32),
          pltpu.VMEM((NBUF, K, DP), jnp.int32),
          pltpu.VMEM((NBUF, K, DP), jnp.int32),
      ] + [pltpu.SemaphoreType.DMA] * (2 * NBUF),
  )(xsp, xtp, src, tgt)


def kernel(x_source, x_target, edge_label_index):
  src = edge_label_index[0].astype(jnp.int32)
  tgt = edge_label_index[1].astype(jnp.int32)
  pad = E_PAD - E
  src = jnp.concatenate([src, jnp.zeros((pad,), jnp.int32)])
  tgt = jnp.concatenate([tgt, jnp.zeros((pad,), jnp.int32)])
  xsp, xtp = _norm_pack(x_source, x_target)
  out = _cosine_sc(xsp, xtp, src, tgt)
  return out[:E]
